# baseline (speedup 1.0000x reference)
import jax
import jax.numpy as jnp
from jax import lax
from jax.experimental import pallas as pl
from jax.experimental.pallas import tpu as pltpu
from jax.experimental.pallas import tpu_sc as plsc

_BLK = 128
_NBUF = 5
_LANES = 16


def _make_lookup(seq, batch, emb_dim):
    info = plsc.get_sparse_core_info()
    nw = info.num_cores * info.num_subcores
    blocks_per_seq = batch // _BLK
    n_chunks = seq * blocks_per_seq
    assert n_chunks % (nw * _NBUF) == 0
    cpw = n_chunks // nw
    niter = cpw // _NBUF
    mesh = plsc.VectorSubcoreMesh(core_axis_name="c", subcore_axis_name="s")

    def body(table_hbm, idx_hbm, out_hbm, idx_v, *bufs):
        rows = bufs[:_NBUF]
        tbuf = bufs[_NBUF : 2 * _NBUF]
        gs = bufs[2 * _NBUF : 3 * _NBUF]
        ws = bufs[3 * _NBUF : 4 * _NBUF]
        wid = lax.axis_index("s") * info.num_cores + lax.axis_index("c")
        chunk0 = wid * cpw
        pltpu.sync_copy(idx_hbm.at[pl.ds(chunk0 * _BLK, cpw * _BLK)], idx_v)

        def gather(j, b):
            pltpu.async_copy(
                table_hbm.at[idx_v.at[pl.ds(j * _BLK, _BLK)]], rows[b], gs[b]
            )

        def wait_gather(j, b):
            pltpu.make_async_copy(
                table_hbm.at[idx_v.at[pl.ds(j * _BLK, _BLK)]], rows[b], gs[b]
            ).wait()

        def wb(j, b):
            c = chunk0 + j
            s = c // blocks_per_seq
            b0 = (c % blocks_per_seq) * _BLK
            pltpu.async_copy(
                tbuf[b], out_hbm.at[s, :, pl.ds(b0, _BLK)], ws[b]
            )

        def wait_wb(b):
            pltpu.make_async_copy(
                tbuf[b], out_hbm.at[0, :, pl.ds(0, _BLK)], ws[b]
            ).wait()

        def transpose(b):
            rb, tb = rows[b], tbuf[b]

            def erow(e, carry):
                col = jnp.full((_LANES,), e, jnp.int32)
                for k in range(_BLK // _LANES):
                    row = lax.iota(jnp.int32, _LANES) + (k * _LANES)
                    tb[e, pl.ds(k * _LANES, _LANES)] = plsc.load_gather(
                        rb, [row, col]
                    )
                return carry

            lax.fori_loop(0, emb_dim, erow, 0)

        for b in range(_NBUF):
            gather(b, b)

        def outer(g, carry):
            for b in range(_NBUF):
                j = g * _NBUF + b
                wait_gather(j, b)

                @pl.when(g >= 1)
                def _():
                    wait_wb(b)

                transpose(b)

                @pl.when(g < niter - 1)
                def _():
                    gather(j + _NBUF, b)

                wb(j, b)
            return carry

        lax.fori_loop(0, niter, outer, 0)
        for b in range(_NBUF):
            wait_wb(b)

    return pl.kernel(
        body,
        out_type=jax.ShapeDtypeStruct((seq, emb_dim, batch), jnp.float32),
        mesh=mesh,
        compiler_params=pltpu.CompilerParams(
            use_tc_tiling_on_sc=False, needs_layout_passes=False
        ),
        scratch_types=(
            [pltpu.VMEM((cpw * _BLK,), jnp.int32)]
            + [pltpu.VMEM((_BLK, emb_dim), jnp.float32) for _ in range(_NBUF)]
            + [pltpu.VMEM((emb_dim, _BLK), jnp.float32) for _ in range(_NBUF)]
            + [pltpu.SemaphoreType.DMA for _ in range(2 * _NBUF)]
        ),
    )


def kernel(x, table):
    b, s = x.shape
    emb_dim = table.shape[1]
    idx_sm = jnp.transpose(x).reshape(b * s)
    out_sm = _make_lookup(s, b, emb_dim)(table, idx_sm)
    return jnp.transpose(out_sm, (2, 0, 1))


# speedup vs baseline: 1.5481x; 1.5481x over previous
import jax
import jax.numpy as jnp
from jax import lax
from jax.experimental import pallas as pl
from jax.experimental.pallas import tpu as pltpu
from jax.experimental.pallas import tpu_sc as plsc

_BLK = 128
_NBUF = 5


def _make_gather(n_chunks, emb_dim):
    info = plsc.get_sparse_core_info()
    nw = info.num_cores * info.num_subcores
    assert n_chunks % (nw * _NBUF) == 0
    cpw = n_chunks // nw
    niter = cpw // _NBUF
    mesh = plsc.VectorSubcoreMesh(core_axis_name="c", subcore_axis_name="s")

    def body(table_hbm, idx_hbm, out_hbm, idx_v, *bufs):
        rows = bufs[:_NBUF]
        gs = bufs[_NBUF : 2 * _NBUF]
        ws = bufs[2 * _NBUF : 3 * _NBUF]
        wid = lax.axis_index("s") * info.num_cores + lax.axis_index("c")
        chunk0 = wid * cpw
        pltpu.sync_copy(idx_hbm.at[pl.ds(chunk0 * _BLK, cpw * _BLK)], idx_v)

        def gather(j, b):
            pltpu.async_copy(
                table_hbm.at[idx_v.at[pl.ds(j * _BLK, _BLK)]], rows[b], gs[b]
            )

        def wait_gather(j, b):
            pltpu.make_async_copy(
                table_hbm.at[idx_v.at[pl.ds(j * _BLK, _BLK)]], rows[b], gs[b]
            ).wait()

        def wait_wb(b):
            pltpu.make_async_copy(
                rows[b], out_hbm.at[:, pl.ds(0, emb_dim)], ws[b]
            ).wait()

        for b in range(_NBUF - 1):
            gather(b, b)

        def outer(g, carry):
            for b in range(_NBUF):
                j = g * _NBUF + b
                p = (b - 1) % _NBUF
                wait_gather(j, b)
                pltpu.async_copy(
                    rows[b],
                    out_hbm.at[:, pl.ds((chunk0 + j) * emb_dim, emb_dim)],
                    ws[b],
                )
                if b == 0:

                    @pl.when(g >= 1)
                    def _():
                        wait_wb(p)

                    gather(j + _NBUF - 1, p)
                else:

                    @pl.when(g <= niter - 2)
                    def _():
                        wait_wb(p)
                        gather(j + _NBUF - 1, p)

            return carry

        lax.fori_loop(0, niter, outer, 0)
        for b in range(_NBUF):
            wait_wb(b)

    return pl.kernel(
        body,
        out_type=jax.ShapeDtypeStruct((_BLK, n_chunks * emb_dim), jnp.float32),
        mesh=mesh,
        compiler_params=pltpu.CompilerParams(
            use_tc_tiling_on_sc=False, needs_layout_passes=False
        ),
        scratch_types=(
            [pltpu.VMEM((cpw * _BLK,), jnp.int32)]
            + [pltpu.VMEM((_BLK, emb_dim), jnp.float32) for _ in range(_NBUF)]
            + [pltpu.SemaphoreType.DMA for _ in range(2 * _NBUF)]
        ),
    )


def _make_transpose(seq, batch, emb_dim):
    blocks_per_seq = batch // _BLK
    n_chunks = seq * blocks_per_seq
    grp = 8
    gb = grp * _BLK

    def body(in_ref, out_ref):
        r = jnp.transpose(in_ref[...])
        for t in range(grp):
            out_ref[0, :, t * _BLK : (t + 1) * _BLK] = r[
                t * emb_dim : (t + 1) * emb_dim, :
            ]

    return pl.pallas_call(
        body,
        grid=(n_chunks // grp,),
        in_specs=[pl.BlockSpec((_BLK, grp * emb_dim), lambda c: (0, c))],
        out_specs=pl.BlockSpec(
            (1, emb_dim, gb),
            lambda c: (c // (blocks_per_seq // grp), 0, c % (blocks_per_seq // grp)),
        ),
        out_shape=jax.ShapeDtypeStruct((seq, emb_dim, batch), jnp.float32),
    )


def kernel(x, table):
    b, s = x.shape
    emb_dim = table.shape[1]
    n_chunks = b * s // _BLK
    idx_sm = jnp.transpose(x).reshape(b * s)
    chunks = _make_gather(n_chunks, emb_dim)(table, idx_sm)
    out_sm = _make_transpose(s, b, emb_dim)(chunks)
    return jnp.transpose(out_sm, (2, 0, 1))
